# Initial kernel scaffold; baseline (speedup 1.0000x reference)
import jax
import jax.numpy as jnp
from jax.experimental import pallas as pl


def kernel(x, edge_index, w1l, b1l, w1r, w2l, b2l, w2r, w3l, b3l, w3r, g1, be1, g2, be2):
    raise NotImplementedError("write your pallas kernel here")


# speedup vs baseline: 7.7820x; 7.7820x over previous
import jax
import jax.numpy as jnp
from jax import lax
from jax.experimental import pallas as pl
from jax.experimental.pallas import tpu as pltpu
from jax.experimental.pallas import tpu_sc as plsc

NC = 2
NS = 16
NW = NC * NS
LANES = 16
CH = 80


def _acc_slices(n):
  rpt = (n // (8 * NS)) * 8
  tail = n - rpt * NS
  assert tail % 8 == 0 and tail <= CH
  return rpt, tail


def _zero_rows(buf, rows, width):
  zero16 = jnp.zeros((LANES,), jnp.float32)

  def zrow(i, _):
    for k in range(width // LANES):
      buf[i, pl.ds(k * LANES, LANES)] = zero16
    return 0

  lax.fori_loop(0, rows, zrow, 0)


def _spread_zero(zbuf, sh, s, n, rpt, tail):
  base = s * rpt
  full, rem = rpt // CH, rpt % CH
  for k in range(full):
    pltpu.sync_copy(zbuf, sh.at[pl.ds(base + k * CH, CH)])
  if rem:
    pltpu.sync_copy(zbuf.at[pl.ds(0, rem)],
                    sh.at[pl.ds(base + full * CH, rem)])
  if tail:
    @pl.when(s == NS - 1)
    def _():
      pltpu.sync_copy(zbuf.at[pl.ds(0, tail)], sh.at[pl.ds(n - tail, tail)])


def _writeback(sh, hbm, c, s, n, rpt, tail):
  base = s * rpt
  pltpu.sync_copy(sh.at[pl.ds(base, rpt)], hbm.at[c, pl.ds(base, rpt)])
  if tail:
    @pl.when(s == NS - 1)
    def _():
      pltpu.sync_copy(sh.at[pl.ds(n - tail, tail)],
                      hbm.at[c, pl.ds(n - tail, tail)])


def _make_seg_sum(n, d, steps):
  rpt, tail = _acc_slices(n)
  mesh = plsc.VectorSubcoreMesh(core_axis_name="c", subcore_axis_name="s")
  scratch = [
      pltpu.VMEM((1, CH), jnp.int32),
      pltpu.VMEM((1, CH), jnp.int32),
      pltpu.VMEM((1, CH), jnp.int32),
      pltpu.VMEM((1, CH), jnp.int32),
      pltpu.VMEM((2, CH, d), jnp.float32),
      pltpu.SemaphoreType.DMA,
      pltpu.SemaphoreType.DMA,
      pltpu.SemaphoreType.DMA,
      pltpu.SemaphoreType.DMA,
      pltpu.VMEM_SHARED((n, d), jnp.float32),
  ]

  def body(h_hbm, src_hbm, dst_hbm, p_hbm, src0, src1, dst0, dst1, rows,
           si0, si1, sg0, sg1, acc_sh):
    c = lax.axis_index("c")
    s = lax.axis_index("s")
    wid = c * NS + s
    srcs = (src0, src1)
    dsts = (dst0, dst1)
    sis = (si0, si1)
    sgs = (sg0, sg1)

    _zero_rows(rows.at[0], CH, d)
    _spread_zero(rows.at[0], acc_sh, s, n, rpt, tail)

    def start_idx(j, sl):
      pltpu.async_copy(src_hbm.at[wid, j], srcs[sl], sis[sl])
      pltpu.async_copy(dst_hbm.at[wid, j], dsts[sl], sis[sl])

    def wait_idx(sl):
      pltpu.make_async_copy(src_hbm.at[wid, 0], srcs[sl], sis[sl]).wait()
      pltpu.make_async_copy(dst_hbm.at[wid, 0], dsts[sl], sis[sl]).wait()

    def start_gather(sl):
      pltpu.async_copy(h_hbm.at[srcs[sl].at[0]], rows.at[sl], sgs[sl])

    def wait_gather(sl):
      pltpu.make_async_copy(h_hbm.at[srcs[sl].at[0]], rows.at[sl],
                            sgs[sl]).wait()

    def scatter(sl):
      pltpu.sync_copy(rows.at[sl], acc_sh.at[dsts[sl].at[0]], add=True)

    plsc.subcore_barrier()

    start_idx(0, 0)
    start_idx(1, 1)
    wait_idx(0)
    start_gather(0)

    def pair(t, _):
      j2 = 2 * t + 2
      j3 = 2 * t + 3
      wait_gather(0)
      wait_idx(1)
      start_gather(1)
      scatter(0)

      @pl.when(j2 < steps)
      def _():
        start_idx(j2, 0)
      wait_gather(1)
      scatter(1)

      @pl.when(j2 < steps)
      def _():
        wait_idx(0)
        start_gather(0)

      @pl.when(j3 < steps)
      def _():
        start_idx(j3, 1)
      return 0

    lax.fori_loop(0, steps // 2, pair, 0)
    if steps % 2:
      wait_gather(0)
      scatter(0)

    plsc.subcore_barrier()
    _writeback(acc_sh, p_hbm, c, s, n, rpt, tail)

  return pl.kernel(body, out_type=jax.ShapeDtypeStruct((NC, n, d),
                                                       jnp.float32),
                   mesh=mesh, scratch_types=scratch)


def _make_counts(n, d, steps):
  rpt, tail = _acc_slices(n)
  mesh = plsc.VectorSubcoreMesh(core_axis_name="c", subcore_axis_name="s")
  scratch = [
      pltpu.VMEM((1, CH), jnp.int32),
      pltpu.VMEM((1, CH), jnp.int32),
      pltpu.VMEM((CH, d), jnp.float32),
      pltpu.SemaphoreType.DMA,
      pltpu.SemaphoreType.DMA,
      pltpu.VMEM_SHARED((n, d), jnp.float32),
  ]

  def body(dst_hbm, cnt_hbm, dst0, dst1, ones_v, si0, si1, cnt_sh):
    c = lax.axis_index("c")
    s = lax.axis_index("s")
    wid = c * NS + s
    dsts = (dst0, dst1)
    sis = (si0, si1)

    _zero_rows(ones_v, CH, d)
    _spread_zero(ones_v, cnt_sh, s, n, rpt, tail)
    onerow = jnp.where(lax.iota(jnp.int32, LANES) == 0, 1.0, 0.0)

    def orow(i, _):
      ones_v[i, pl.ds(0, LANES)] = onerow
      return 0

    lax.fori_loop(0, CH, orow, 0)

    def start_idx(j, sl):
      pltpu.async_copy(dst_hbm.at[wid, j], dsts[sl], sis[sl])

    def wait_idx(sl):
      pltpu.make_async_copy(dst_hbm.at[wid, 0], dsts[sl], sis[sl]).wait()

    def scatter(sl):
      pltpu.sync_copy(ones_v, cnt_sh.at[dsts[sl].at[0]], add=True)

    plsc.subcore_barrier()

    start_idx(0, 0)
    start_idx(1, 1)

    def pair(t, _):
      j2 = 2 * t + 2
      j3 = 2 * t + 3
      wait_idx(0)
      scatter(0)

      @pl.when(j2 < steps)
      def _():
        start_idx(j2, 0)
      wait_idx(1)
      scatter(1)

      @pl.when(j3 < steps)
      def _():
        start_idx(j3, 1)
      return 0

    lax.fori_loop(0, steps // 2, pair, 0)
    if steps % 2:
      wait_idx(0)
      scatter(0)

    plsc.subcore_barrier()
    _writeback(cnt_sh, cnt_hbm, c, s, n, rpt, tail)

  return pl.kernel(body, out_type=jax.ShapeDtypeStruct((NC, n, d),
                                                       jnp.float32),
                   mesh=mesh, scratch_types=scratch)


def _make_tc_layer(n, d, relu_bn, residual, blk=1000):
  grid = n // blk
  bnscale = 1.0 / float(1.0 + 1e-5) ** 0.5

  def body(*refs):
    if residual:
      p_ref, cnt_ref, h_ref, wl_ref, wr_ref, b_ref, g_ref, be_ref, r_ref, \
          o_ref = refs
    elif relu_bn:
      p_ref, cnt_ref, h_ref, wl_ref, wr_ref, b_ref, g_ref, be_ref, o_ref = \
          refs
    else:
      p_ref, cnt_ref, h_ref, wl_ref, wr_ref, b_ref, o_ref = refs
    cnt = cnt_ref[0, :, 0:1] + cnt_ref[1, :, 0:1]
    inv = 1.0 / jnp.maximum(cnt, 1.0)
    agg = (p_ref[0] + p_ref[1]) * inv
    y = (jnp.dot(agg, wl_ref[...], preferred_element_type=jnp.float32)
         + b_ref[...]
         + jnp.dot(h_ref[...], wr_ref[...],
                   preferred_element_type=jnp.float32))
    if relu_bn:
      y = jnp.maximum(y, 0.0)
      y = y * (g_ref[...] * bnscale) + be_ref[...]
    if residual:
      y = y + r_ref[...]
    o_ref[...] = y

  w_spec = pl.BlockSpec((d, d), lambda i: (0, 0))
  vec_spec = pl.BlockSpec((1, d), lambda i: (0, 0))
  in_specs = [
      pl.BlockSpec((NC, blk, d), lambda i: (0, i, 0)),
      pl.BlockSpec((NC, blk, d), lambda i: (0, i, 0)),
      pl.BlockSpec((blk, d), lambda i: (i, 0)),
      w_spec, w_spec, vec_spec,
  ]
  if relu_bn:
    in_specs += [vec_spec, vec_spec]
  if residual:
    in_specs.append(pl.BlockSpec((blk, d), lambda i: (i, 0)))

  return pl.pallas_call(
      body,
      grid=(grid,),
      in_specs=in_specs,
      out_specs=pl.BlockSpec((blk, d), lambda i: (i, 0)),
      out_shape=jax.ShapeDtypeStruct((n, d), jnp.float32),
  )


def kernel(x, edge_index, w1l, b1l, w1r, w2l, b2l, w2r, w3l, b3l, w3r,
           g1, be1, g2, be2):
  n, d = x.shape
  e = edge_index.shape[1]
  steps = e // (NW * CH)
  src3d = edge_index[0].reshape(NW, steps, 1, CH)
  dst3d = edge_index[1].reshape(NW, steps, 1, CH)

  seg = _make_seg_sum(n, d, steps)
  counts = _make_counts(n, d, steps)
  tc_rb = _make_tc_layer(n, d, True, False)
  tc_res = _make_tc_layer(n, d, True, True)
  tc_plain = _make_tc_layer(n, d, False, False)

  cnt = counts(dst3d)
  p = seg(x, src3d, dst3d)
  x1 = tc_rb(p, cnt, x, w1l.T, w1r.T, b1l.reshape(1, d),
             g1.reshape(1, d), be1.reshape(1, d))
  p = seg(x1, src3d, dst3d)
  x2 = tc_res(p, cnt, x1, w2l.T, w2r.T, b2l.reshape(1, d),
              g2.reshape(1, d), be2.reshape(1, d), x1)
  p = seg(x2, src3d, dst3d)
  x3 = tc_plain(p, cnt, x2, w3l.T, w3r.T, b3l.reshape(1, d))
  return x3


# speedup vs baseline: 10.1232x; 1.3008x over previous
import jax
import jax.numpy as jnp
from jax import lax
from jax.experimental import pallas as pl
from jax.experimental.pallas import tpu as pltpu
from jax.experimental.pallas import tpu_sc as plsc

NC = 2
NS = 16
NW = NC * NS
LANES = 16
CH = 125


def _acc_slices(n):
  rpt = (n // (8 * NS)) * 8
  tail = n - rpt * NS
  assert tail % 8 == 0 and tail <= CH
  return rpt, tail


def _zero_rows(buf, rows, width):
  zero16 = jnp.zeros((LANES,), jnp.float32)

  def zrow(i, _):
    for k in range(width // LANES):
      buf[i, pl.ds(k * LANES, LANES)] = zero16
    return 0

  lax.fori_loop(0, rows, zrow, 0)


def _spread_zero(zbuf, sh, s, n, rpt, tail):
  base = s * rpt
  zch = (CH // 8) * 8
  full, rem = rpt // zch, rpt % zch
  for k in range(full):
    pltpu.sync_copy(zbuf.at[pl.ds(0, zch)], sh.at[pl.ds(base + k * zch, zch)])
  if rem:
    pltpu.sync_copy(zbuf.at[pl.ds(0, rem)],
                    sh.at[pl.ds(base + full * zch, rem)])
  if tail:
    @pl.when(s == NS - 1)
    def _():
      pltpu.sync_copy(zbuf.at[pl.ds(0, tail)], sh.at[pl.ds(n - tail, tail)])


def _writeback(sh, hbm, c, s, n, rpt, tail):
  base = s * rpt
  pltpu.sync_copy(sh.at[pl.ds(base, rpt)], hbm.at[c, pl.ds(base, rpt)])
  if tail:
    @pl.when(s == NS - 1)
    def _():
      pltpu.sync_copy(sh.at[pl.ds(n - tail, tail)],
                      hbm.at[c, pl.ds(n - tail, tail)])


def _make_seg_sum(n, d, steps):
  rpt, tail = _acc_slices(n)
  mesh = plsc.VectorSubcoreMesh(core_axis_name="c", subcore_axis_name="s")
  scratch = [
      pltpu.VMEM((1, CH), jnp.int32),
      pltpu.VMEM((1, CH), jnp.int32),
      pltpu.VMEM((1, CH), jnp.int32),
      pltpu.VMEM((1, CH), jnp.int32),
      pltpu.VMEM((2, CH, d), jnp.float32),
      pltpu.SemaphoreType.DMA,
      pltpu.SemaphoreType.DMA,
      pltpu.SemaphoreType.DMA,
      pltpu.SemaphoreType.DMA,
      pltpu.VMEM_SHARED((n, d), jnp.float32),
  ]

  def body(h_hbm, src_hbm, dst_hbm, p_hbm, src0, src1, dst0, dst1, rows,
           si0, si1, sg0, sg1, acc_sh):
    c = lax.axis_index("c")
    s = lax.axis_index("s")
    wid = c * NS + s
    srcs = (src0, src1)
    dsts = (dst0, dst1)
    sis = (si0, si1)
    sgs = (sg0, sg1)

    _zero_rows(rows.at[0], CH, d)
    _spread_zero(rows.at[0], acc_sh, s, n, rpt, tail)

    def start_idx(j, sl):
      pltpu.async_copy(src_hbm.at[wid, j], srcs[sl], sis[sl])
      pltpu.async_copy(dst_hbm.at[wid, j], dsts[sl], sis[sl])

    def wait_idx(sl):
      pltpu.make_async_copy(src_hbm.at[wid, 0], srcs[sl], sis[sl]).wait()
      pltpu.make_async_copy(dst_hbm.at[wid, 0], dsts[sl], sis[sl]).wait()

    def start_gather(sl):
      pltpu.async_copy(h_hbm.at[srcs[sl].at[0]], rows.at[sl], sgs[sl])

    def wait_gather(sl):
      pltpu.make_async_copy(h_hbm.at[srcs[sl].at[0]], rows.at[sl],
                            sgs[sl]).wait()

    def scatter(sl):
      pltpu.sync_copy(rows.at[sl], acc_sh.at[dsts[sl].at[0]], add=True)

    plsc.subcore_barrier()

    start_idx(0, 0)
    start_idx(1, 1)
    wait_idx(0)
    start_gather(0)

    def pair(t, _):
      j2 = 2 * t + 2
      j3 = 2 * t + 3
      wait_gather(0)
      wait_idx(1)
      start_gather(1)
      scatter(0)

      @pl.when(j2 < steps)
      def _():
        start_idx(j2, 0)
      wait_gather(1)

      @pl.when(j2 < steps)
      def _():
        wait_idx(0)
        start_gather(0)
      scatter(1)

      @pl.when(j3 < steps)
      def _():
        start_idx(j3, 1)
      return 0

    lax.fori_loop(0, steps // 2, pair, 0)
    if steps % 2:
      wait_gather(0)
      scatter(0)

    plsc.subcore_barrier()
    _writeback(acc_sh, p_hbm, c, s, n, rpt, tail)

  return pl.kernel(body, out_type=jax.ShapeDtypeStruct((NC, n, d),
                                                       jnp.float32),
                   mesh=mesh, scratch_types=scratch)


def _make_counts(n, d, steps):
  rpt, tail = _acc_slices(n)
  mesh = plsc.VectorSubcoreMesh(core_axis_name="c", subcore_axis_name="s")
  scratch = [
      pltpu.VMEM((1, CH), jnp.int32),
      pltpu.VMEM((1, CH), jnp.int32),
      pltpu.VMEM((CH, d), jnp.float32),
      pltpu.SemaphoreType.DMA,
      pltpu.SemaphoreType.DMA,
      pltpu.VMEM_SHARED((n, d), jnp.float32),
  ]

  def body(dst_hbm, cnt_hbm, dst0, dst1, ones_v, si0, si1, cnt_sh):
    c = lax.axis_index("c")
    s = lax.axis_index("s")
    wid = c * NS + s
    dsts = (dst0, dst1)
    sis = (si0, si1)

    _zero_rows(ones_v, CH, d)
    _spread_zero(ones_v, cnt_sh, s, n, rpt, tail)
    onerow = jnp.where(lax.iota(jnp.int32, LANES) == 0, 1.0, 0.0)

    def orow(i, _):
      ones_v[i, pl.ds(0, LANES)] = onerow
      return 0

    lax.fori_loop(0, CH, orow, 0)

    def start_idx(j, sl):
      pltpu.async_copy(dst_hbm.at[wid, j], dsts[sl], sis[sl])

    def wait_idx(sl):
      pltpu.make_async_copy(dst_hbm.at[wid, 0], dsts[sl], sis[sl]).wait()

    def scatter(sl):
      pltpu.sync_copy(ones_v, cnt_sh.at[dsts[sl].at[0]], add=True)

    plsc.subcore_barrier()

    start_idx(0, 0)
    start_idx(1, 1)

    def pair(t, _):
      j2 = 2 * t + 2
      j3 = 2 * t + 3
      wait_idx(0)
      scatter(0)

      @pl.when(j2 < steps)
      def _():
        start_idx(j2, 0)
      wait_idx(1)
      scatter(1)

      @pl.when(j3 < steps)
      def _():
        start_idx(j3, 1)
      return 0

    lax.fori_loop(0, steps // 2, pair, 0)
    if steps % 2:
      wait_idx(0)
      scatter(0)

    plsc.subcore_barrier()
    _writeback(cnt_sh, cnt_hbm, c, s, n, rpt, tail)

  return pl.kernel(body, out_type=jax.ShapeDtypeStruct((NC, n, d),
                                                       jnp.float32),
                   mesh=mesh, scratch_types=scratch)


def _make_tc_layer(n, d, relu_bn, residual, blk=1000):
  grid = n // blk
  bnscale = 1.0 / float(1.0 + 1e-5) ** 0.5

  def body(*refs):
    if residual:
      p_ref, cnt_ref, h_ref, wl_ref, wr_ref, b_ref, g_ref, be_ref, r_ref, \
          o_ref = refs
    elif relu_bn:
      p_ref, cnt_ref, h_ref, wl_ref, wr_ref, b_ref, g_ref, be_ref, o_ref = \
          refs
    else:
      p_ref, cnt_ref, h_ref, wl_ref, wr_ref, b_ref, o_ref = refs
    cnt = cnt_ref[0, :, 0:1] + cnt_ref[1, :, 0:1]
    inv = 1.0 / jnp.maximum(cnt, 1.0)
    agg = (p_ref[0] + p_ref[1]) * inv
    y = (jnp.dot(agg, wl_ref[...], preferred_element_type=jnp.float32)
         + b_ref[...]
         + jnp.dot(h_ref[...], wr_ref[...],
                   preferred_element_type=jnp.float32))
    if relu_bn:
      y = jnp.maximum(y, 0.0)
      y = y * (g_ref[...] * bnscale) + be_ref[...]
    if residual:
      y = y + r_ref[...]
    o_ref[...] = y

  w_spec = pl.BlockSpec((d, d), lambda i: (0, 0))
  vec_spec = pl.BlockSpec((1, d), lambda i: (0, 0))
  in_specs = [
      pl.BlockSpec((NC, blk, d), lambda i: (0, i, 0)),
      pl.BlockSpec((NC, blk, d), lambda i: (0, i, 0)),
      pl.BlockSpec((blk, d), lambda i: (i, 0)),
      w_spec, w_spec, vec_spec,
  ]
  if relu_bn:
    in_specs += [vec_spec, vec_spec]
  if residual:
    in_specs.append(pl.BlockSpec((blk, d), lambda i: (i, 0)))

  return pl.pallas_call(
      body,
      grid=(grid,),
      in_specs=in_specs,
      out_specs=pl.BlockSpec((blk, d), lambda i: (i, 0)),
      out_shape=jax.ShapeDtypeStruct((n, d), jnp.float32),
  )


def kernel(x, edge_index, w1l, b1l, w1r, w2l, b2l, w2r, w3l, b3l, w3r,
           g1, be1, g2, be2):
  n, d = x.shape
  e = edge_index.shape[1]
  steps = e // (NW * CH)
  src3d = edge_index[0].reshape(NW, steps, 1, CH)
  dst3d = edge_index[1].reshape(NW, steps, 1, CH)

  seg = _make_seg_sum(n, d, steps)
  counts = _make_counts(n, d, steps)
  tc_rb = _make_tc_layer(n, d, True, False)
  tc_res = _make_tc_layer(n, d, True, True)
  tc_plain = _make_tc_layer(n, d, False, False)

  cnt = counts(dst3d)
  p = seg(x, src3d, dst3d)
  x1 = tc_rb(p, cnt, x, w1l.T, w1r.T, b1l.reshape(1, d),
             g1.reshape(1, d), be1.reshape(1, d))
  p = seg(x1, src3d, dst3d)
  x2 = tc_res(p, cnt, x1, w2l.T, w2r.T, b2l.reshape(1, d),
              g2.reshape(1, d), be2.reshape(1, d), x1)
  p = seg(x2, src3d, dst3d)
  x3 = tc_plain(p, cnt, x2, w3l.T, w3r.T, b3l.reshape(1, d))
  return x3


# speedup vs baseline: 11.1755x; 1.1039x over previous
import jax
import jax.numpy as jnp
from jax import lax
from jax.experimental import pallas as pl
from jax.experimental.pallas import tpu as pltpu
from jax.experimental.pallas import tpu_sc as plsc

NC = 2
NS = 16
NW = NC * NS
LANES = 16
CH = 125


def _acc_slices(n):
  rpt = (n // (8 * NS)) * 8
  tail = n - rpt * NS
  assert tail % 8 == 0 and tail <= CH
  return rpt, tail


def _zero_rows(buf, rows, width):
  zero16 = jnp.zeros((LANES,), jnp.float32)

  def zrow(i, _):
    for k in range(width // LANES):
      buf[i, pl.ds(k * LANES, LANES)] = zero16
    return 0

  lax.fori_loop(0, rows, zrow, 0)


def _spread_zero(zbuf, sh, s, n, rpt, tail):
  base = s * rpt
  zch = (CH // 8) * 8
  full, rem = rpt // zch, rpt % zch
  for k in range(full):
    pltpu.sync_copy(zbuf.at[pl.ds(0, zch)], sh.at[pl.ds(base + k * zch, zch)])
  if rem:
    pltpu.sync_copy(zbuf.at[pl.ds(0, rem)],
                    sh.at[pl.ds(base + full * zch, rem)])
  if tail:
    @pl.when(s == NS - 1)
    def _():
      pltpu.sync_copy(zbuf.at[pl.ds(0, tail)], sh.at[pl.ds(n - tail, tail)])


def _writeback(sh, hbm, c, s, n, rpt, tail):
  base = s * rpt
  pltpu.sync_copy(sh.at[pl.ds(base, rpt)], hbm.at[c, pl.ds(base, rpt)])
  if tail:
    @pl.when(s == NS - 1)
    def _():
      pltpu.sync_copy(sh.at[pl.ds(n - tail, tail)],
                      hbm.at[c, pl.ds(n - tail, tail)])


def _make_seg_sum(n, d, steps):
  rpt, tail = _acc_slices(n)
  mesh = plsc.VectorSubcoreMesh(core_axis_name="c", subcore_axis_name="s")
  scratch = [
      pltpu.VMEM((1, CH), jnp.int32),
      pltpu.VMEM((1, CH), jnp.int32),
      pltpu.VMEM((1, CH), jnp.int32),
      pltpu.VMEM((1, CH), jnp.int32),
      pltpu.VMEM((2, CH, d), jnp.float32),
      pltpu.SemaphoreType.DMA,
      pltpu.SemaphoreType.DMA,
      pltpu.SemaphoreType.DMA,
      pltpu.SemaphoreType.DMA,
      pltpu.VMEM_SHARED((n, d), jnp.float32),
  ]

  def body(h_hbm, src_hbm, dst_hbm, p_hbm, src0, src1, dst0, dst1, rows,
           si0, si1, sg0, sg1, acc_sh):
    c = lax.axis_index("c")
    s = lax.axis_index("s")
    wid = c * NS + s
    srcs = (src0, src1)
    dsts = (dst0, dst1)
    sis = (si0, si1)
    sgs = (sg0, sg1)

    _zero_rows(rows.at[0], CH, d)
    _spread_zero(rows.at[0], acc_sh, s, n, rpt, tail)

    def start_idx(j, sl):
      pltpu.async_copy(src_hbm.at[wid, j], srcs[sl], sis[sl])
      pltpu.async_copy(dst_hbm.at[wid, j], dsts[sl], sis[sl])

    def wait_idx(sl):
      pltpu.make_async_copy(src_hbm.at[wid, 0], srcs[sl], sis[sl]).wait()
      pltpu.make_async_copy(dst_hbm.at[wid, 0], dsts[sl], sis[sl]).wait()

    def start_gather(sl):
      pltpu.async_copy(h_hbm.at[srcs[sl].at[0]], rows.at[sl], sgs[sl])

    def wait_gather(sl):
      pltpu.make_async_copy(h_hbm.at[srcs[sl].at[0]], rows.at[sl],
                            sgs[sl]).wait()

    def scatter(sl):
      pltpu.sync_copy(rows.at[sl], acc_sh.at[dsts[sl].at[0]], add=True)

    plsc.subcore_barrier()

    start_idx(0, 0)
    start_idx(1, 1)
    wait_idx(0)
    start_gather(0)

    def pair(t, _):
      j2 = 2 * t + 2
      j3 = 2 * t + 3
      wait_gather(0)
      wait_idx(1)
      start_gather(1)
      scatter(0)

      @pl.when(j2 < steps)
      def _():
        start_idx(j2, 0)
      wait_gather(1)

      @pl.when(j2 < steps)
      def _():
        wait_idx(0)
        start_gather(0)
      scatter(1)

      @pl.when(j3 < steps)
      def _():
        start_idx(j3, 1)
      return 0

    lax.fori_loop(0, steps // 2, pair, 0)
    if steps % 2:
      wait_gather(0)
      scatter(0)

    plsc.subcore_barrier()
    _writeback(acc_sh, p_hbm, c, s, n, rpt, tail)

  return pl.kernel(body, out_type=jax.ShapeDtypeStruct((NC, n, d),
                                                       jnp.float32),
                   mesh=mesh, scratch_types=scratch)


def _make_counts(n, steps):
  fr = -(-n // 128)
  fr = -(-fr // 8) * 8
  cw = 128
  mesh = plsc.VectorSubcoreMesh(core_axis_name="c", subcore_axis_name="s")
  scratch = [
      pltpu.VMEM((1, cw), jnp.int32),
      pltpu.VMEM((1, cw), jnp.int32),
      pltpu.VMEM((fr, 128), jnp.float32),
      pltpu.VMEM((1, fr), jnp.int32),
      pltpu.SemaphoreType.DMA,
      pltpu.SemaphoreType.DMA,
      pltpu.VMEM_SHARED((fr, 128), jnp.float32),
  ]

  def body(dst_hbm, cnt_hbm, dst0, dst1, loc, iota_v, si0, si1, cnt_sh):
    c = lax.axis_index("c")
    s = lax.axis_index("s")
    wid = c * NS + s
    dsts = (dst0, dst1)
    sis = (si0, si1)

    _zero_rows(loc, fr, 128)

    @pl.when(s == 0)
    def _():
      pltpu.sync_copy(loc, cnt_sh)
    iota16 = lax.iota(jnp.int32, LANES)
    for k in range(fr // LANES):
      iota_v[0, pl.ds(k * LANES, LANES)] = iota16 + (LANES * k)
    ones16 = jnp.full((LANES,), 1.0, jnp.float32)

    def start_idx(j, sl):
      pltpu.async_copy(dst_hbm.at[wid, j], dsts[sl], sis[sl])

    def wait_idx(sl):
      pltpu.make_async_copy(dst_hbm.at[wid, 0], dsts[sl], sis[sl]).wait()

    def count(sl):
      b = dsts[sl]
      for m in range(cw // LANES):
        v = b[0, pl.ds(m * LANES, LANES)]
        r = lax.shift_right_logical(v, 7)
        cc = lax.bitwise_and(v, 127)
        plsc.addupdate_scatter(loc, [r, cc], ones16)

    plsc.subcore_barrier()
    start_idx(0, 0)
    start_idx(1, 1)

    def pair(t, _):
      j2 = 2 * t + 2
      j3 = 2 * t + 3
      wait_idx(0)
      count(0)

      @pl.when(j2 < steps)
      def _():
        start_idx(j2, 0)
      wait_idx(1)
      count(1)

      @pl.when(j3 < steps)
      def _():
        start_idx(j3, 1)
      return 0

    lax.fori_loop(0, steps // 2, pair, 0)
    if steps % 2:
      wait_idx(0)
      count(0)

    pltpu.sync_copy(loc, cnt_sh.at[iota_v.at[0]], add=True)
    plsc.subcore_barrier()

    @pl.when(s == 0)
    def _():
      pltpu.sync_copy(cnt_sh, cnt_hbm.at[c])

  return pl.kernel(
      body,
      out_type=jax.ShapeDtypeStruct((NC, fr, 128), jnp.float32),
      mesh=mesh, scratch_types=scratch,
      compiler_params=pltpu.CompilerParams(needs_layout_passes=False))


def _make_inv(n):
  fr = -(-n // 128)
  fr = -(-fr // 8) * 8

  def body(cnt_ref, o_ref):
    o_ref[...] = 1.0 / jnp.maximum(cnt_ref[0] + cnt_ref[1], 1.0)

  return pl.pallas_call(
      body,
      grid=(1,),
      in_specs=[pl.BlockSpec((NC, fr, 128), lambda i: (0, 0, 0))],
      out_specs=pl.BlockSpec((fr, 128), lambda i: (0, 0)),
      out_shape=jax.ShapeDtypeStruct((fr, 128), jnp.float32),
  )


def _make_tc_layer(n, d, relu_bn, residual, blk=1000):
  grid = n // blk
  bnscale = 1.0 / float(1.0 + 1e-5) ** 0.5

  def body(*refs):
    if residual:
      p_ref, cnt_ref, h_ref, wl_ref, wr_ref, b_ref, g_ref, be_ref, r_ref, \
          o_ref = refs
    elif relu_bn:
      p_ref, cnt_ref, h_ref, wl_ref, wr_ref, b_ref, g_ref, be_ref, o_ref = \
          refs
    else:
      p_ref, cnt_ref, h_ref, wl_ref, wr_ref, b_ref, o_ref = refs
    agg = (p_ref[0] + p_ref[1]) * cnt_ref[...]
    y = (jnp.dot(agg, wl_ref[...], preferred_element_type=jnp.float32)
         + b_ref[...]
         + jnp.dot(h_ref[...], wr_ref[...],
                   preferred_element_type=jnp.float32))
    if relu_bn:
      y = jnp.maximum(y, 0.0)
      y = y * (g_ref[...] * bnscale) + be_ref[...]
    if residual:
      y = y + r_ref[...]
    o_ref[...] = y

  w_spec = pl.BlockSpec((d, d), lambda i: (0, 0))
  vec_spec = pl.BlockSpec((1, d), lambda i: (0, 0))
  in_specs = [
      pl.BlockSpec((NC, blk, d), lambda i: (0, i, 0)),
      pl.BlockSpec((blk, 1), lambda i: (i, 0)),
      pl.BlockSpec((blk, d), lambda i: (i, 0)),
      w_spec, w_spec, vec_spec,
  ]
  if relu_bn:
    in_specs += [vec_spec, vec_spec]
  if residual:
    in_specs.append(pl.BlockSpec((blk, d), lambda i: (i, 0)))

  return pl.pallas_call(
      body,
      grid=(grid,),
      in_specs=in_specs,
      out_specs=pl.BlockSpec((blk, d), lambda i: (i, 0)),
      out_shape=jax.ShapeDtypeStruct((n, d), jnp.float32),
  )


def kernel(x, edge_index, w1l, b1l, w1r, w2l, b2l, w2r, w3l, b3l, w3r,
           g1, be1, g2, be2):
  n, d = x.shape
  e = edge_index.shape[1]
  steps = e // (NW * CH)
  src3d = edge_index[0].reshape(NW, steps, 1, CH)
  dst3d = edge_index[1].reshape(NW, steps, 1, CH)

  seg = _make_seg_sum(n, d, steps)
  counts = _make_counts(n, steps)
  tc_rb = _make_tc_layer(n, d, True, False)
  tc_res = _make_tc_layer(n, d, True, True)
  tc_plain = _make_tc_layer(n, d, False, False)

  dstp = jnp.pad(dst3d, ((0, 0), (0, 0), (0, 0), (0, 128 - CH)),
                 constant_values=n)
  cntf = counts(dstp)
  inv = _make_inv(n)(cntf).reshape(-1)[:n].reshape(n, 1)
  p = seg(x, src3d, dst3d)
  x1 = tc_rb(p, inv, x, w1l.T, w1r.T, b1l.reshape(1, d),
             g1.reshape(1, d), be1.reshape(1, d))
  p = seg(x1, src3d, dst3d)
  x2 = tc_res(p, inv, x1, w2l.T, w2r.T, b2l.reshape(1, d),
              g2.reshape(1, d), be2.reshape(1, d), x1)
  p = seg(x2, src3d, dst3d)
  x3 = tc_plain(p, inv, x2, w3l.T, w3r.T, b3l.reshape(1, d))
  return x3
